# Initial kernel scaffold; baseline (speedup 1.0000x reference)
import jax
import jax.numpy as jnp
from jax import lax
from jax.experimental import pallas as pl
from jax.experimental.pallas import tpu as pltpu
from jax.experimental.pallas import tpu_sc as plsc

NC = 2
NS = 16
CH = 80


def _sc_gather_segsum(x, src, dst, with_counts):
    N, D = x.shape
    E = src.shape[0]
    NW = NC * NS
    epw = E // NW
    nchunk = epw // CH
    rps = N // NS

    mesh = plsc.VectorSubcoreMesh(core_axis_name="c", subcore_axis_name="s")
    zf = jnp.zeros((N, D), jnp.float32)
    if with_counts:
        zc = jnp.zeros((N, 16), jnp.float32)
        out_type = (jax.ShapeDtypeStruct((NC, N, D), jnp.float32),
                    jax.ShapeDtypeStruct((NC, N, 16), jnp.float32))
    else:
        out_type = jax.ShapeDtypeStruct((NC, N, D), jnp.float32)

    scratch = [pltpu.VMEM((CH,), jnp.int32),
               pltpu.VMEM((CH,), jnp.int32),
               pltpu.VMEM((CH, D), jnp.float32),
               pltpu.VMEM_SHARED((N, D), jnp.float32),
               pltpu.SemaphoreType.DMA]
    if with_counts:
        scratch += [pltpu.VMEM((CH, 16), jnp.float32),
                    pltpu.VMEM_SHARED((N, 16), jnp.float32)]

    def body(*refs):
        if with_counts:
            (x_hbm, src_hbm, dst_hbm, zf_hbm, zc_hbm, sum_hbm, cnt_hbm,
             src_v, dst_v, rows_v, acc_sh, sem, ones_v, cnt_sh) = refs
        else:
            (x_hbm, src_hbm, dst_hbm, zf_hbm, sum_hbm,
             src_v, dst_v, rows_v, acc_sh, sem) = refs
        c = lax.axis_index("c")
        s = lax.axis_index("s")
        wid = s * NC + c
        row0 = s * rps

        pltpu.sync_copy(zf_hbm.at[pl.ds(row0, rps)], acc_sh.at[pl.ds(row0, rps)])
        if with_counts:
            pltpu.sync_copy(zc_hbm.at[pl.ds(row0, rps)],
                            cnt_sh.at[pl.ds(row0, rps)])

            @pl.loop(0, CH)
            def _(i):
                ones_v[i, :] = jnp.ones((16,), jnp.float32)

        plsc.subcore_barrier()

        base = wid * epw

        @pl.loop(0, nchunk)
        def _(k):
            off = base + k * CH
            pltpu.sync_copy(src_hbm.at[pl.ds(off, CH)], src_v)
            pltpu.sync_copy(dst_hbm.at[pl.ds(off, CH)], dst_v)
            pltpu.async_copy(x_hbm.at[src_v], rows_v, sem).wait()
            pltpu.sync_copy(rows_v, acc_sh.at[dst_v], add=True)
            if with_counts:
                pltpu.sync_copy(ones_v, cnt_sh.at[dst_v], add=True)

        plsc.subcore_barrier()
        pltpu.sync_copy(acc_sh.at[pl.ds(row0, rps)],
                        sum_hbm.at[c, pl.ds(row0, rps)])
        if with_counts:
            pltpu.sync_copy(cnt_sh.at[pl.ds(row0, rps)],
                            cnt_hbm.at[c, pl.ds(row0, rps)])

    kern = pl.kernel(body, out_type=out_type, mesh=mesh, scratch_types=scratch)
    if with_counts:
        return kern(x, src, dst, zf, zc)
    return kern(x, src, dst, zf)


_ROWS = 1000


def _tc_layer1(sum_pair, cnt_pair, x, WlT, bl, WrT):
    N, D = x.shape
    H = WlT.shape[1]
    R = _ROWS

    def body(sum_ref, cnt_ref, x_ref, wl_ref, bl_ref, wr_ref, o_ref):
        ssum = sum_ref[0] + sum_ref[1]
        cnt = cnt_ref[0][:, 0:1] + cnt_ref[1][:, 0:1]
        mean = ssum / jnp.maximum(cnt, 1.0)
        h = (jnp.dot(mean, wl_ref[...], preferred_element_type=jnp.float32)
             + bl_ref[...]
             + jnp.dot(x_ref[...], wr_ref[...], preferred_element_type=jnp.float32))
        o_ref[...] = jnp.maximum(h, 0.0)

    return pl.pallas_call(
        body,
        grid=(N // R,),
        in_specs=[pl.BlockSpec((NC, R, D), lambda i: (0, i, 0)),
                  pl.BlockSpec((NC, R, 16), lambda i: (0, i, 0)),
                  pl.BlockSpec((R, D), lambda i: (i, 0)),
                  pl.BlockSpec((D, H), lambda i: (0, 0)),
                  pl.BlockSpec((1, H), lambda i: (0, 0)),
                  pl.BlockSpec((D, H), lambda i: (0, 0))],
        out_specs=pl.BlockSpec((R, H), lambda i: (i, 0)),
        out_shape=jax.ShapeDtypeStruct((N, H), jnp.float32),
    )(sum_pair, cnt_pair, x, WlT, bl.reshape(1, H), WrT)


def _tc_layer2_heads(sum_pair, cnt_pair, h, WlT, bl, WrT, WaT, ba, WcT, bc):
    N, H = h.shape
    R = _ROWS
    G = N // R

    def body(sum_ref, cnt_ref, h_ref, wl_ref, bl_ref, wr_ref,
             wa_ref, ba_ref, wc_ref, bc_ref,
             actor_ref, csum_ref, critic_ref):
        i = pl.program_id(0)
        ssum = sum_ref[0] + sum_ref[1]
        cnt = cnt_ref[0][:, 0:1] + cnt_ref[1][:, 0:1]
        mean = ssum / jnp.maximum(cnt, 1.0)
        h2 = jnp.maximum(
            jnp.dot(mean, wl_ref[...], preferred_element_type=jnp.float32)
            + bl_ref[...]
            + jnp.dot(h_ref[...], wr_ref[...], preferred_element_type=jnp.float32),
            0.0)
        actor_ref[...] = (jnp.dot(h2, wa_ref[...],
                                  preferred_element_type=jnp.float32)
                          + ba_ref[...])
        part = jnp.sum(h2, axis=0, keepdims=True)

        @pl.when(i == 0)
        def _():
            csum_ref[...] = part

        @pl.when(i > 0)
        def _():
            csum_ref[...] = csum_ref[...] + part

        @pl.when(i == G - 1)
        def _():
            critic_ref[...] = (jnp.dot(csum_ref[...] / N, wc_ref[...],
                                       preferred_element_type=jnp.float32)
                               + bc_ref[...])

    return pl.pallas_call(
        body,
        grid=(G,),
        in_specs=[pl.BlockSpec((NC, R, H), lambda i: (0, i, 0)),
                  pl.BlockSpec((NC, R, 16), lambda i: (0, i, 0)),
                  pl.BlockSpec((R, H), lambda i: (i, 0)),
                  pl.BlockSpec((H, H), lambda i: (0, 0)),
                  pl.BlockSpec((1, H), lambda i: (0, 0)),
                  pl.BlockSpec((H, H), lambda i: (0, 0)),
                  pl.BlockSpec((H, 1), lambda i: (0, 0)),
                  pl.BlockSpec((1, 1), lambda i: (0, 0)),
                  pl.BlockSpec((H, 1), lambda i: (0, 0)),
                  pl.BlockSpec((1, 1), lambda i: (0, 0))],
        out_specs=[pl.BlockSpec((R, 1), lambda i: (i, 0)),
                   pl.BlockSpec((1, H), lambda i: (0, 0)),
                   pl.BlockSpec((1, 1), lambda i: (0, 0))],
        out_shape=[jax.ShapeDtypeStruct((N, 1), jnp.float32),
                   jax.ShapeDtypeStruct((1, H), jnp.float32),
                   jax.ShapeDtypeStruct((1, 1), jnp.float32)],
    )(sum_pair, cnt_pair, h, WlT, bl.reshape(1, H), WrT,
      WaT, ba.reshape(1, 1), WcT, bc.reshape(1, 1))


def kernel(x, edge_index, Wl1, bl1, Wr1, Wl2, bl2, Wr2, Wa, ba, Wc, bc):
    src = edge_index[0]
    dst = edge_index[1]
    sum1, cnt = _sc_gather_segsum(x, src, dst, True)
    h = _tc_layer1(sum1, cnt, x, Wl1.T, bl1, Wr1.T)
    sum2 = _sc_gather_segsum(h, src, dst, False)
    actor, _, critic = _tc_layer2_heads(sum2, cnt, h, Wl2.T, bl2, Wr2.T,
                                        Wa.T, ba, Wc.T, bc)
    return actor[:, 0], critic[0, 0]


# speedup vs baseline: 7.9013x; 7.9013x over previous
import jax
import jax.numpy as jnp
from jax import lax
from jax.experimental import pallas as pl
from jax.experimental.pallas import tpu as pltpu
from jax.experimental.pallas import tpu_sc as plsc

NC = 2
NS = 16
SB = 64
CH = 128

_SC_MESH = plsc.VectorSubcoreMesh(core_axis_name="c", subcore_axis_name="s")


def _node_pad(n):
    return -(-n // (NS * SB)) * (NS * SB)


def _sc_segsum(x, src2, dst2):
    N, D = x.shape
    E = src2.shape[1]
    NP = _node_pad(N)
    rps = NP // NS
    nsb = rps // SB
    grid = E // CH
    zf = jnp.zeros((SB, D), jnp.float32)

    def body(x_hbm, src_hbm, dst_hbm, zf_hbm, sum_hbm, rows_v, stf_v, acc_sh):
        c = lax.axis_index("c")
        s = lax.axis_index("s")
        row0 = s * rps

        pltpu.sync_copy(zf_hbm, stf_v)

        @pl.loop(0, nsb)
        def _(j):
            pltpu.sync_copy(stf_v, acc_sh.at[pl.ds(row0 + j * SB, SB)])

        plsc.subcore_barrier()

        def chunk(src_blk, dst_blk):
            pltpu.sync_copy(x_hbm.at[src_blk.at[0]], rows_v)
            pltpu.sync_copy(rows_v, acc_sh.at[dst_blk.at[0]], add=True)

        pltpu.emit_pipeline(
            chunk,
            grid=(grid,),
            in_specs=[pl.BlockSpec((1, CH), lambda i: (0, i)),
                      pl.BlockSpec((1, CH), lambda i: (0, i))],
            out_specs=[],
            core_axis_name=("c", "s"),
            dimension_semantics=(pltpu.PARALLEL,),
        )(src_hbm, dst_hbm)

        plsc.subcore_barrier()

        @pl.loop(0, nsb)
        def _(j):
            r = row0 + j * SB
            pltpu.sync_copy(acc_sh.at[pl.ds(r, SB)], stf_v)
            pltpu.sync_copy(stf_v, sum_hbm.at[c, pl.ds(r, SB)])

    kern = pl.kernel(
        body,
        out_type=jax.ShapeDtypeStruct((NC, NP, D), jnp.float32),
        mesh=_SC_MESH,
        scratch_types=[pltpu.VMEM((CH, D), jnp.float32),
                       pltpu.VMEM((SB, D), jnp.float32),
                       pltpu.VMEM_SHARED((NP, D), jnp.float32)])
    return kern(x, src2, dst2, zf)


def _sc_counts(dst2, N, D):
    E = dst2.shape[1]
    NP = _node_pad(N)
    rps = NP // NS
    nsb = rps // SB
    grid = E // CH
    zc = jnp.zeros((SB, D), jnp.float32)
    on = jnp.ones((CH, D), jnp.float32)

    def body(dst_hbm, zc_hbm, on_hbm, cnt_hbm, stc_v, ones_v, cnt_sh):
        c = lax.axis_index("c")
        s = lax.axis_index("s")
        row0 = s * rps

        pltpu.sync_copy(zc_hbm, stc_v)
        pltpu.sync_copy(on_hbm, ones_v)

        @pl.loop(0, nsb)
        def _(j):
            pltpu.sync_copy(stc_v, cnt_sh.at[pl.ds(row0 + j * SB, SB)])

        plsc.subcore_barrier()

        def chunk(dst_blk):
            pltpu.sync_copy(ones_v, cnt_sh.at[dst_blk.at[0]], add=True)

        pltpu.emit_pipeline(
            chunk,
            grid=(grid,),
            in_specs=[pl.BlockSpec((1, CH), lambda i: (0, i))],
            out_specs=[],
            core_axis_name=("c", "s"),
            dimension_semantics=(pltpu.PARALLEL,),
        )(dst_hbm)

        plsc.subcore_barrier()

        @pl.loop(0, nsb)
        def _(j):
            r = row0 + j * SB
            pltpu.sync_copy(cnt_sh.at[pl.ds(r, SB)], stc_v)
            pltpu.sync_copy(stc_v, cnt_hbm.at[c, pl.ds(r, SB)])

    kern = pl.kernel(
        body,
        out_type=jax.ShapeDtypeStruct((NC, NP, D), jnp.float32),
        mesh=_SC_MESH,
        scratch_types=[pltpu.VMEM((SB, D), jnp.float32),
                       pltpu.VMEM((CH, D), jnp.float32),
                       pltpu.VMEM_SHARED((NP, D), jnp.float32)])
    return kern(dst2, zc, on)


_ROWS = 1000


def _tc_layer1(sum_pair, cnt_pair, x, WlT, bl, WrT):
    N, D = x.shape
    H = WlT.shape[1]
    R = _ROWS

    def body(sum_ref, cnt_ref, x_ref, wl_ref, bl_ref, wr_ref, o_ref):
        ssum = sum_ref[0] + sum_ref[1]
        cnt = cnt_ref[0][:, 0:1] + cnt_ref[1][:, 0:1]
        mean = ssum / jnp.maximum(cnt, 1.0)
        h = (jnp.dot(mean, wl_ref[...], preferred_element_type=jnp.float32)
             + bl_ref[...]
             + jnp.dot(x_ref[...], wr_ref[...], preferred_element_type=jnp.float32))
        o_ref[...] = jnp.maximum(h, 0.0)

    return pl.pallas_call(
        body,
        grid=(N // R,),
        in_specs=[pl.BlockSpec((NC, R, D), lambda i: (0, i, 0)),
                  pl.BlockSpec((NC, R, D), lambda i: (0, i, 0)),
                  pl.BlockSpec((R, D), lambda i: (i, 0)),
                  pl.BlockSpec((D, H), lambda i: (0, 0)),
                  pl.BlockSpec((1, H), lambda i: (0, 0)),
                  pl.BlockSpec((D, H), lambda i: (0, 0))],
        out_specs=pl.BlockSpec((R, H), lambda i: (i, 0)),
        out_shape=jax.ShapeDtypeStruct((N, H), jnp.float32),
    )(sum_pair, cnt_pair, x, WlT, bl.reshape(1, H), WrT)


def _tc_layer2_heads(sum_pair, cnt_pair, h, WlT, bl, WrT, WaT, ba, WcT, bc):
    N, H = h.shape
    R = _ROWS
    G = N // R

    def body(sum_ref, cnt_ref, h_ref, wl_ref, bl_ref, wr_ref,
             wa_ref, ba_ref, wc_ref, bc_ref,
             actor_ref, csum_ref, critic_ref):
        i = pl.program_id(0)
        ssum = sum_ref[0] + sum_ref[1]
        cnt = cnt_ref[0][:, 0:1] + cnt_ref[1][:, 0:1]
        mean = ssum / jnp.maximum(cnt, 1.0)
        h2 = jnp.maximum(
            jnp.dot(mean, wl_ref[...], preferred_element_type=jnp.float32)
            + bl_ref[...]
            + jnp.dot(h_ref[...], wr_ref[...], preferred_element_type=jnp.float32),
            0.0)
        actor_ref[...] = (jnp.dot(h2, wa_ref[...],
                                  preferred_element_type=jnp.float32)
                          + ba_ref[...])
        part = jnp.sum(h2, axis=0, keepdims=True)

        @pl.when(i == 0)
        def _():
            csum_ref[...] = part

        @pl.when(i > 0)
        def _():
            csum_ref[...] = csum_ref[...] + part

        @pl.when(i == G - 1)
        def _():
            critic_ref[...] = (jnp.dot(csum_ref[...] / N, wc_ref[...],
                                       preferred_element_type=jnp.float32)
                               + bc_ref[...])

    return pl.pallas_call(
        body,
        grid=(G,),
        in_specs=[pl.BlockSpec((NC, R, H), lambda i: (0, i, 0)),
                  pl.BlockSpec((NC, R, H), lambda i: (0, i, 0)),
                  pl.BlockSpec((R, H), lambda i: (i, 0)),
                  pl.BlockSpec((H, H), lambda i: (0, 0)),
                  pl.BlockSpec((1, H), lambda i: (0, 0)),
                  pl.BlockSpec((H, H), lambda i: (0, 0)),
                  pl.BlockSpec((H, 1), lambda i: (0, 0)),
                  pl.BlockSpec((1, 1), lambda i: (0, 0)),
                  pl.BlockSpec((H, 1), lambda i: (0, 0)),
                  pl.BlockSpec((1, 1), lambda i: (0, 0))],
        out_specs=[pl.BlockSpec((R, 1), lambda i: (i, 0)),
                   pl.BlockSpec((1, H), lambda i: (0, 0)),
                   pl.BlockSpec((1, 1), lambda i: (0, 0))],
        out_shape=[jax.ShapeDtypeStruct((N, 1), jnp.float32),
                   jax.ShapeDtypeStruct((1, H), jnp.float32),
                   jax.ShapeDtypeStruct((1, 1), jnp.float32)],
    )(sum_pair, cnt_pair, h, WlT, bl.reshape(1, H), WrT,
      WaT, ba.reshape(1, 1), WcT, bc.reshape(1, 1))


def kernel(x, edge_index, Wl1, bl1, Wr1, Wl2, bl2, Wr2, Wa, ba, Wc, bc):
    src2 = edge_index[0:1]
    dst2 = edge_index[1:2]
    cnt = _sc_counts(dst2, x.shape[0], x.shape[1])
    sum1 = _sc_segsum(x, src2, dst2)
    h = _tc_layer1(sum1, cnt, x, Wl1.T, bl1, Wr1.T)
    sum2 = _sc_segsum(h, src2, dst2)
    actor, _, critic = _tc_layer2_heads(sum2, cnt, h, Wl2.T, bl2, Wr2.T,
                                        Wa.T, ba, Wc.T, bc)
    return actor[:, 0], critic[0, 0]
